# Initial kernel scaffold; baseline (speedup 1.0000x reference)
import jax
import jax.numpy as jnp
from jax.experimental import pallas as pl


def kernel(tokens, embed_table, W_out, b_out):
    raise NotImplementedError("write your pallas kernel here")


# speedup vs baseline: 3.1813x; 3.1813x over previous
import functools

import jax
import jax.numpy as jnp
from jax import lax
from jax.experimental import pallas as pl
from jax.experimental.pallas import tpu as pltpu
from jax.experimental.pallas import tpu_sc as plsc

E = 64
B_TOK = 4096
L_TOK = 200
N_IDX = B_TOK * L_TOK

_info = plsc.get_sparse_core_info()
NC, NS = _info.num_cores, _info.num_subcores
NW = NC * NS
CHUNK = 128
B_PER_W = N_IDX // NW
N_CHUNKS = B_PER_W // CHUNK


def _proj_body(tab_ref, w_ref, b_ref, out_ref):
    out_ref[...] = (
        jnp.dot(tab_ref[...], w_ref[...], preferred_element_type=jnp.float32)
        + b_ref[...]
    )


def _project_table(embed_table, W_t, b_row):
    rows = embed_table.shape[0]
    blk = 2048
    grid = (rows + blk - 1) // blk
    return pl.pallas_call(
        _proj_body,
        grid=(grid,),
        in_specs=[
            pl.BlockSpec((blk, E), lambda i: (i, 0)),
            pl.BlockSpec((E, E), lambda i: (0, 0)),
            pl.BlockSpec((1, E), lambda i: (0, 0)),
        ],
        out_specs=pl.BlockSpec((blk, E), lambda i: (i, 0)),
        out_shape=jax.ShapeDtypeStruct((rows, E), jnp.float32),
    )(embed_table, W_t, b_row)


def _gather_body(table_hbm, idx_hbm, out_hbm, idx_v, buf_v, gsem, osem):
    wid = lax.axis_index("s") * NC + lax.axis_index("c")
    pltpu.sync_copy(idx_hbm.at[wid], idx_v)

    def body(j, carry):
        pltpu.async_copy(table_hbm.at[idx_v.at[j]], buf_v, gsem).wait()
        pltpu.async_copy(buf_v, out_hbm.at[wid, j], osem).wait()
        return carry

    lax.fori_loop(0, N_CHUNKS, body, 0)


@functools.partial(jax.jit, static_argnums=())
def _sc_gather(table, idx3):
    mesh = plsc.VectorSubcoreMesh(core_axis_name="c", subcore_axis_name="s")
    f = pl.kernel(
        _gather_body,
        mesh=mesh,
        compiler_params=pltpu.CompilerParams(use_tc_tiling_on_sc=False),
        out_type=jax.ShapeDtypeStruct((NW, N_CHUNKS, CHUNK, E), jnp.float32),
        scratch_types=[
            pltpu.VMEM((N_CHUNKS, CHUNK), jnp.int32),
            pltpu.VMEM((CHUNK, E), jnp.float32),
            pltpu.SemaphoreType.DMA,
            pltpu.SemaphoreType.DMA,
        ],
    )
    return f(table, idx3)


def kernel(tokens, embed_table, W_out, b_out):
    idx3 = tokens.astype(jnp.int32).reshape(NW, N_CHUNKS, CHUNK)
    proj = _project_table(embed_table, W_out.T, b_out.reshape(1, E))
    out = _sc_gather(proj, idx3)
    return out.reshape(B_TOK, L_TOK, 1, E)


# speedup vs baseline: 3.7573x; 1.1811x over previous
import functools

import jax
import jax.numpy as jnp
from jax import lax
from jax.experimental import pallas as pl
from jax.experimental.pallas import tpu as pltpu
from jax.experimental.pallas import tpu_sc as plsc

E = 64
B_TOK = 4096
L_TOK = 200
N_IDX = B_TOK * L_TOK

_info = plsc.get_sparse_core_info()
NC, NS = _info.num_cores, _info.num_subcores
NW = NC * NS
CHUNK = 128
B_PER_W = N_IDX // NW
N_CHUNKS = B_PER_W // CHUNK


def _proj_body(tab_ref, w_ref, b_ref, out_ref):
    out_ref[...] = (
        jnp.dot(tab_ref[...], w_ref[...], preferred_element_type=jnp.float32)
        + b_ref[...]
    )


def _project_table(embed_table, W_t, b_row):
    rows = embed_table.shape[0]
    blk = 2048
    grid = (rows + blk - 1) // blk
    return pl.pallas_call(
        _proj_body,
        grid=(grid,),
        in_specs=[
            pl.BlockSpec((blk, E), lambda i: (i, 0)),
            pl.BlockSpec((E, E), lambda i: (0, 0)),
            pl.BlockSpec((1, E), lambda i: (0, 0)),
        ],
        out_specs=pl.BlockSpec((blk, E), lambda i: (i, 0)),
        out_shape=jax.ShapeDtypeStruct((rows, E), jnp.float32),
    )(embed_table, W_t, b_row)


NBUF = 4


def _gather_body(table_hbm, idx_hbm, out_hbm, idx_v, buf_v, gsem, osem):
    wid = lax.axis_index("s") * NC + lax.axis_index("c")
    base = wid * B_PER_W
    pltpu.sync_copy(idx_hbm.at[wid], idx_v)

    def start_gather(j, b):
        pltpu.async_copy(table_hbm.at[idx_v.at[j]], buf_v.at[b], gsem.at[b])

    def wait_gather(j, b):
        pltpu.make_async_copy(
            table_hbm.at[idx_v.at[j]], buf_v.at[b], gsem.at[b]
        ).wait()

    def start_wb(j, b):
        pltpu.async_copy(
            buf_v.at[b], out_hbm.at[pl.ds(base + j * CHUNK, CHUNK)], osem.at[b]
        )

    def wait_wb(j, b):
        pltpu.make_async_copy(
            buf_v.at[b], out_hbm.at[pl.ds(base + j * CHUNK, CHUNK)], osem.at[b]
        ).wait()

    for b in range(NBUF):
        start_gather(b, b)

    def body(j, carry):
        b = lax.rem(j, NBUF)
        wait_gather(j, b)
        start_wb(j, b)

        @pl.when(jnp.logical_and(j >= 1, j + NBUF - 1 < N_CHUNKS))
        def _():
            pb = lax.rem(j - 1, NBUF)
            wait_wb(j - 1, pb)
            start_gather(j + NBUF - 1, pb)

        return carry

    lax.fori_loop(0, N_CHUNKS, body, 0)

    for j in range(N_CHUNKS - NBUF, N_CHUNKS):
        wait_wb(j, j % NBUF)


@functools.partial(jax.jit, static_argnums=())
def _sc_gather(table, idx3):
    mesh = plsc.VectorSubcoreMesh(core_axis_name="c", subcore_axis_name="s")
    f = pl.kernel(
        _gather_body,
        mesh=mesh,
        compiler_params=pltpu.CompilerParams(use_tc_tiling_on_sc=False),
        out_type=jax.ShapeDtypeStruct((N_IDX, E), jnp.float32),
        scratch_types=[
            pltpu.VMEM((N_CHUNKS, CHUNK), jnp.int32),
            pltpu.VMEM((NBUF, CHUNK, E), jnp.float32),
            pltpu.SemaphoreType.DMA((NBUF,)),
            pltpu.SemaphoreType.DMA((NBUF,)),
        ],
    )
    return f(table, idx3)


def kernel(tokens, embed_table, W_out, b_out):
    idx3 = tokens.astype(jnp.int32).reshape(NW, N_CHUNKS, CHUNK)
    proj = _project_table(embed_table, W_out.T, b_out.reshape(1, E))
    out = _sc_gather(proj, idx3)
    return out.reshape(B_TOK, L_TOK, 1, E)
